# Initial kernel scaffold; baseline (speedup 1.0000x reference)
import jax
import jax.numpy as jnp
from jax.experimental import pallas as pl


def kernel(token_embeddings, pos_table):
    raise NotImplementedError("write your pallas kernel here")


# speedup vs baseline: 2.0855x; 2.0855x over previous
import jax
import jax.numpy as jnp
from jax.experimental import pallas as pl

_BS = 512


def _ln_add_block(x_ref, pos_ref, o_ref):
    x = x_ref[0]
    mean = jnp.mean(x, axis=-1, keepdims=True)
    xc = x - mean
    var = jnp.mean(xc * xc, axis=-1, keepdims=True)
    o_ref[0] = xc * jax.lax.rsqrt(var + 1e-5) + pos_ref[...]


def kernel(token_embeddings, pos_table):
    b, s, d = token_embeddings.shape
    grid = (s // _BS, b)
    return pl.pallas_call(
        _ln_add_block,
        grid=grid,
        in_specs=[
            pl.BlockSpec((1, _BS, d), lambda i, j: (j, i, 0)),
            pl.BlockSpec((_BS, d), lambda i, j: (i, 0)),
        ],
        out_specs=pl.BlockSpec((1, _BS, d), lambda i, j: (j, i, 0)),
        out_shape=jax.ShapeDtypeStruct((b, s, d), token_embeddings.dtype),
    )(token_embeddings, pos_table[:s])


# speedup vs baseline: 2.3395x; 1.1218x over previous
import jax
import jax.numpy as jnp
from jax.experimental import pallas as pl

_BS = 1024


def _ln_add_block(x_ref, pos_ref, o_ref):
    x = x_ref[0]
    mean = jnp.mean(x, axis=-1, keepdims=True)
    xc = x - mean
    var = jnp.mean(xc * xc, axis=-1, keepdims=True)
    o_ref[0] = xc * jax.lax.rsqrt(var + 1e-5) + pos_ref[...]


def kernel(token_embeddings, pos_table):
    b, s, d = token_embeddings.shape
    grid = (s // _BS, b)
    return pl.pallas_call(
        _ln_add_block,
        grid=grid,
        in_specs=[
            pl.BlockSpec((1, _BS, d), lambda i, j: (j, i, 0)),
            pl.BlockSpec((_BS, d), lambda i, j: (i, 0)),
        ],
        out_specs=pl.BlockSpec((1, _BS, d), lambda i, j: (j, i, 0)),
        out_shape=jax.ShapeDtypeStruct((b, s, d), token_embeddings.dtype),
    )(token_embeddings, pos_table[:s])


# speedup vs baseline: 2.4708x; 1.0561x over previous
import jax
import jax.numpy as jnp
from jax.experimental import pallas as pl

_BS = 2048


def _ln_add_block(x_ref, pos_ref, o_ref):
    x = x_ref[0]
    mean = jnp.mean(x, axis=-1, keepdims=True)
    xc = x - mean
    var = jnp.mean(xc * xc, axis=-1, keepdims=True)
    o_ref[0] = xc * jax.lax.rsqrt(var + 1e-5) + pos_ref[...]


def kernel(token_embeddings, pos_table):
    b, s, d = token_embeddings.shape
    grid = (s // _BS, b)
    return pl.pallas_call(
        _ln_add_block,
        grid=grid,
        in_specs=[
            pl.BlockSpec((1, _BS, d), lambda i, j: (j, i, 0)),
            pl.BlockSpec((_BS, d), lambda i, j: (i, 0)),
        ],
        out_specs=pl.BlockSpec((1, _BS, d), lambda i, j: (j, i, 0)),
        out_shape=jax.ShapeDtypeStruct((b, s, d), token_embeddings.dtype),
    )(token_embeddings, pos_table[:s])
